# baseline (speedup 1.0000x reference)
import functools

import jax
import jax.numpy as jnp
from jax import lax
from jax.experimental import pallas as pl
from jax.experimental.pallas import tpu as pltpu
from jax.experimental.pallas import tpu_sc as plsc

N_NODES = 10000
N_EDGES = 320000
D_FEAT = 128

NC = 2
NS = 16
NW = NC * NS
EPW = N_EDGES // NW
B = 80
NCHUNK = EPW // B
L = 16
GPC = B // L
D = 2


@functools.partial(
    pl.kernel,
    mesh=plsc.VectorSubcoreMesh(core_axis_name="c", subcore_axis_name="s"),
    out_type=jax.ShapeDtypeStruct((N_EDGES,), jnp.float32),
    scratch_types=[
        pltpu.VMEM((EPW,), jnp.int32),
        pltpu.VMEM((EPW,), jnp.int32),
        pltpu.VMEM((D, B, D_FEAT), jnp.float32),
        pltpu.VMEM((D, B, D_FEAT), jnp.float32),
        pltpu.VMEM((EPW,), jnp.float32),
        pltpu.SemaphoreType.DMA,
        pltpu.SemaphoreType.DMA,
    ],
)
def _edge_dot(h_hbm, src_hbm, dst_hbm, out_hbm, idx_s, idx_d, u_buf, v_buf,
              out_v, sem0, sem1):
    wid = lax.axis_index("s") * NC + lax.axis_index("c")
    base = wid * EPW
    sems = (sem0, sem1)

    pltpu.sync_copy(src_hbm.at[pl.ds(base, EPW)], idx_s)
    pltpu.sync_copy(dst_hbm.at[pl.ds(base, EPW)], idx_d)

    lanes = lax.iota(jnp.int32, L)

    def start(c, s):
        off = c * B
        pltpu.async_copy(h_hbm.at[idx_s.at[pl.ds(off, B)]], u_buf.at[s],
                         sems[s])
        pltpu.async_copy(h_hbm.at[idx_d.at[pl.ds(off, B)]], v_buf.at[s],
                         sems[s])

    def wait_slot(c, s):
        off = c * B
        pltpu.make_async_copy(h_hbm.at[idx_s.at[pl.ds(off, B)]], u_buf.at[s],
                              sems[s]).wait()
        pltpu.make_async_copy(h_hbm.at[idx_d.at[pl.ds(off, B)]], v_buf.at[s],
                              sems[s]).wait()

    def compute(c, s):
        off = c * B

        def group_body(g, carry2):
            rbase = g * L
            svec = jnp.zeros((L,), jnp.float32)
            for e in range(L):
                r = rbase + e
                prods = [u_buf[s, r, pl.ds(k * L, L)]
                         * v_buf[s, r, pl.ds(k * L, L)]
                         for k in range(D_FEAT // L)]
                while len(prods) > 1:
                    prods = [prods[i] + prods[i + 1]
                             for i in range(0, len(prods), 2)]
                acc = prods[0]
                for sh in (8, 4, 2, 1):
                    acc = acc + acc.at[(lanes + sh) % L].get(
                        mode="promise_in_bounds")
                svec = jnp.where(lanes == e, acc, svec)
            out_v[pl.ds(off + g * L, L)] = svec
            return carry2

        lax.fori_loop(0, GPC, group_body, jnp.int32(0))

    for j in range(D - 1):
        start(j, j)

    def block_body(ib, carry):
        c0 = ib * D
        for k in range(D):
            c = c0 + k

            @pl.when(c + D - 1 < NCHUNK)
            def _():
                start(c + D - 1, (k + D - 1) % D)

            wait_slot(c, k)
            compute(c, k)
        return carry

    n_blocks = (NCHUNK - 1) // D
    lax.fori_loop(0, n_blocks, block_body, jnp.int32(0))

    last = NCHUNK - 1
    wait_slot(last, last % D)
    compute(last, last % D)

    pltpu.sync_copy(out_v, out_hbm.at[pl.ds(base, EPW)])


def kernel(h, edge_index):
    ei = edge_index.astype(jnp.int32)
    scores = _edge_dot(h, ei[0], ei[1])
    return scores.reshape(N_EDGES, 1)


# speedup vs baseline: 1.3592x; 1.3592x over previous
import functools

import jax
import jax.numpy as jnp
from jax import lax
from jax.experimental import pallas as pl
from jax.experimental.pallas import tpu as pltpu
from jax.experimental.pallas import tpu_sc as plsc

N_NODES = 10000
N_EDGES = 320000
D_FEAT = 128

NC = 2
NS = 16
NW = NC * NS
EPW = N_EDGES // NW
B = 80
NCHUNK = EPW // B
L = 16
GPC = B // L
D = 2


@functools.partial(
    pl.kernel,
    mesh=plsc.VectorSubcoreMesh(core_axis_name="c", subcore_axis_name="s"),
    out_type=jax.ShapeDtypeStruct((N_EDGES,), jnp.float32),
    scratch_types=[
        pltpu.VMEM((EPW,), jnp.int32),
        pltpu.VMEM((EPW,), jnp.int32),
        pltpu.VMEM((D, B, D_FEAT), jnp.float32),
        pltpu.VMEM((D, B, D_FEAT), jnp.float32),
        pltpu.VMEM((EPW,), jnp.float32),
        pltpu.SemaphoreType.DMA,
        pltpu.SemaphoreType.DMA,
    ],
)
def _edge_dot(h_hbm, src_hbm, dst_hbm, out_hbm, idx_s, idx_d, u_buf, v_buf,
              out_v, sem0, sem1):
    wid = lax.axis_index("s") * NC + lax.axis_index("c")
    base = wid * EPW
    sems = (sem0, sem1)

    pltpu.sync_copy(src_hbm.at[pl.ds(base, EPW)], idx_s)
    pltpu.sync_copy(dst_hbm.at[pl.ds(base, EPW)], idx_d)

    lanes = lax.iota(jnp.int32, L)

    def start(c, s):
        off = c * B
        pltpu.async_copy(h_hbm.at[idx_s.at[pl.ds(off, B)]], u_buf.at[s],
                         sems[s])
        pltpu.async_copy(h_hbm.at[idx_d.at[pl.ds(off, B)]], v_buf.at[s],
                         sems[s])

    def wait_slot(c, s):
        off = c * B
        pltpu.make_async_copy(h_hbm.at[idx_s.at[pl.ds(off, B)]], u_buf.at[s],
                              sems[s]).wait()
        pltpu.make_async_copy(h_hbm.at[idx_d.at[pl.ds(off, B)]], v_buf.at[s],
                              sems[s]).wait()

    def compute(c, s):
        off = c * B

        def group_body(g, carry2):
            rbase = g * L
            svec = jnp.zeros((L,), jnp.float32)
            for e in range(L):
                r = rbase + e
                acc0 = u_buf[s, r, pl.ds(0, L)] * v_buf[s, r, pl.ds(0, L)]
                acc1 = u_buf[s, r, pl.ds(L, L)] * v_buf[s, r, pl.ds(L, L)]
                for k in range(2, D_FEAT // L, 2):
                    acc0 = acc0 + (u_buf[s, r, pl.ds(k * L, L)]
                                   * v_buf[s, r, pl.ds(k * L, L)])
                    acc1 = acc1 + (u_buf[s, r, pl.ds((k + 1) * L, L)]
                                   * v_buf[s, r, pl.ds((k + 1) * L, L)])
                acc = acc0 + acc1
                for sh in (8, 4, 2, 1):
                    acc = acc + acc.at[(lanes + sh) % L].get(
                        mode="promise_in_bounds")
                svec = jnp.where(lanes == e, acc, svec)
            out_v[pl.ds(off + g * L, L)] = svec
            return carry2

        lax.fori_loop(0, GPC, group_body, jnp.int32(0))

    for j in range(D - 1):
        start(j, j)

    def block_body(ib, carry):
        c0 = ib * D
        for k in range(D):
            c = c0 + k

            @pl.when(c + D - 1 < NCHUNK)
            def _():
                start(c + D - 1, (k + D - 1) % D)

            wait_slot(c, k)
            compute(c, k)
        return carry

    n_blocks = (NCHUNK - 1) // D
    lax.fori_loop(0, n_blocks, block_body, jnp.int32(0))

    last = NCHUNK - 1
    wait_slot(last, last % D)
    compute(last, last % D)

    pltpu.sync_copy(out_v, out_hbm.at[pl.ds(base, EPW)])


def kernel(h, edge_index):
    ei = edge_index.astype(jnp.int32)
    scores = _edge_dot(h, ei[0], ei[1])
    return scores.reshape(N_EDGES, 1)
